# Initial kernel scaffold; baseline (speedup 1.0000x reference)
import jax
import jax.numpy as jnp
from jax.experimental import pallas as pl


def kernel(x, up_rows, up_cols, up_vals, indices, W1, b1, W_d3, b_d3, W_2d3, b_2d3, W_full, b_full):
    raise NotImplementedError("write your pallas kernel here")


# speedup vs baseline: 7.4730x; 7.4730x over previous
import functools

import jax
import jax.numpy as jnp
from jax import lax
from jax.experimental import pallas as pl
from jax.experimental.pallas import tpu as pltpu
from jax.experimental.pallas import tpu_sc as plsc

B = 4
N_LOW = 8192
N_HIGH = 16384
C_IN = 128
C_OUT = 64
J = 9
NNZ = 3 * N_HIGH

NC, NS, LANES = 2, 16, 16
NW = NC * NS
ROWS_PER_W = N_HIGH // NW
CH = 128
GCH = 128
N_GCH = ROWS_PER_W // GCH

_mesh = lambda: plsc.VectorSubcoreMesh(core_axis_name="c", subcore_axis_name="s")


def _pool_body(x2_hbm, cols_hbm, rows_hbm, vals_hbm, bounds_hbm, out_hbm,
               boundsv, colv, rowv, valv, xg, acc, sem):
    wid = lax.axis_index("s") * NC + lax.axis_index("c")
    base = wid * ROWS_PER_W
    pltpu.sync_copy(bounds_hbm, boundsv)
    s = boundsv[pl.ds(wid, LANES)][0]
    e = boundsv[pl.ds(wid + 1, LANES)][0]
    a = (s // 8) * 8
    nch = (e - a + CH - 1) // CH

    for b in range(B):
        def zero_body(r, _):
            for cb in range(C_IN // LANES):
                acc[r, pl.ds(cb * LANES, LANES)] = jnp.zeros((LANES,), jnp.float32)
            return 0
        lax.fori_loop(0, ROWS_PER_W, zero_body, 0)

        col_off = b * N_LOW

        def chunk_body(ci, _):
            off = a + ci * CH
            pltpu.sync_copy(cols_hbm.at[pl.ds(off, CH)], colv)
            pltpu.sync_copy(rows_hbm.at[pl.ds(off, CH)], rowv.at[pl.ds(0, CH)])
            pltpu.sync_copy(vals_hbm.at[pl.ds(off, CH)], valv.at[pl.ds(0, CH)])
            for q in range(CH // LANES):
                sl = pl.ds(q * LANES, LANES)
                colv[sl] = colv[sl] + col_off
            pltpu.async_copy(x2_hbm.at[colv], xg, sem).wait()

            lo = jnp.maximum(s - off, 0)
            hi = jnp.minimum(e - off, CH)

            def k_body(k, _):
                dst = rowv[pl.ds(k, LANES)][0] - base
                vv = jnp.full((LANES,), valv[pl.ds(k, LANES)][0], jnp.float32)
                for cb in range(C_IN // LANES):
                    sl = pl.ds(cb * LANES, LANES)
                    acc[dst, sl] = acc[dst, sl] + xg[k, sl] * vv
                return 0
            lax.fori_loop(lo, hi, k_body, 0)
            return 0
        lax.fori_loop(0, nch, chunk_body, 0)

        pltpu.sync_copy(acc, out_hbm.at[b, pl.ds(base, ROWS_PER_W)])


def _pool(x2, cols, rows, vals, bounds):
    k = pl.kernel(
        _pool_body,
        out_type=jax.ShapeDtypeStruct((B, N_HIGH, C_IN), jnp.float32),
        mesh=_mesh(),
        scratch_types=[
            pltpu.VMEM((NW + LANES + 8,), jnp.int32),
            pltpu.VMEM((CH,), jnp.int32),
            pltpu.VMEM((CH + LANES,), jnp.int32),
            pltpu.VMEM((CH + LANES,), jnp.float32),
            pltpu.VMEM((CH, C_IN), jnp.float32),
            pltpu.VMEM((ROWS_PER_W, C_IN), jnp.float32),
            pltpu.SemaphoreType.DMA,
        ],
    )
    return k(x2, cols, rows, vals, bounds)


def _mm_body(p_ref, w_ref, o_ref):
    o_ref[0, 0] = lax.dot_general(
        p_ref[0], w_ref[0], (((1,), (1,)), ((), ())),
        preferred_element_type=jnp.float32)


def _matmul(pooled, wcat, bt=512):
    return pl.pallas_call(
        _mm_body,
        grid=(B, N_HIGH // bt, J),
        in_specs=[
            pl.BlockSpec((1, bt, C_IN), lambda b, t, j: (b, t, 0)),
            pl.BlockSpec((1, C_OUT, C_IN), lambda b, t, j: (j, 0, 0)),
        ],
        out_specs=pl.BlockSpec((1, 1, bt, C_OUT), lambda b, t, j: (b, j, t, 0)),
        out_shape=jax.ShapeDtypeStruct((B, J, N_HIGH, C_OUT), jnp.float32),
    )(pooled, wcat)


def _spiral_body(h2_hbm, idxt_hbm, bias_hbm, out_hbm,
                 biasv, idxfull, idxv, gbuf, ybuf, sem):
    wid = lax.axis_index("s") * NC + lax.axis_index("c")
    nb = wid * ROWS_PER_W
    pltpu.sync_copy(bias_hbm, biasv)
    for j in range(J):
        pltpu.sync_copy(idxt_hbm.at[pl.ds(j * N_HIGH + nb, ROWS_PER_W)],
                        idxfull.at[pl.ds(j * ROWS_PER_W, ROWS_PER_W)])

    for b in range(B):
        for c in range(N_GCH):
            for j in range(J):
                off = (b * J + j) * N_HIGH
                for q in range(GCH // LANES):
                    dsl = pl.ds(j * GCH + q * LANES, LANES)
                    ssl = pl.ds(j * ROWS_PER_W + c * GCH + q * LANES, LANES)
                    idxv[dsl] = idxfull[ssl] + off
            copies = [
                pltpu.async_copy(h2_hbm.at[idxv.at[pl.ds(j * GCH, GCH)]],
                                 gbuf.at[j], sem)
                for j in range(J)
            ]
            for cp in copies:
                cp.wait()

            def r_body(r, _):
                for cb in range(C_OUT // LANES):
                    sl = pl.ds(cb * LANES, LANES)
                    accv = biasv[sl]
                    for j in range(J):
                        accv = accv + gbuf[j, r, sl]
                    ybuf[r, sl] = jnp.maximum(accv, 0.0)
                return 0
            lax.fori_loop(0, GCH, r_body, 0)

            pltpu.sync_copy(ybuf, out_hbm.at[b, pl.ds(nb + c * GCH, GCH)])


def _spiral(h2, idxt, bias):
    k = pl.kernel(
        _spiral_body,
        out_type=jax.ShapeDtypeStruct((B, N_HIGH, C_OUT), jnp.float32),
        mesh=_mesh(),
        scratch_types=[
            pltpu.VMEM((C_OUT,), jnp.float32),
            pltpu.VMEM((J * ROWS_PER_W,), jnp.int32),
            pltpu.VMEM((J * GCH,), jnp.int32),
            pltpu.VMEM((J, GCH, C_OUT), jnp.float32),
            pltpu.VMEM((GCH, C_OUT), jnp.float32),
            pltpu.SemaphoreType.DMA,
        ],
        compiler_params=pltpu.CompilerParams(use_tc_tiling_on_sc=False),
    )
    return k(h2, idxt, bias)


def kernel(x, up_rows, up_cols, up_vals, indices,
           W1, b1, W_d3, b_d3, W_2d3, b_2d3, W_full, b_full):
    wf = W_full.reshape(C_OUT // 2, J, C_IN).transpose(1, 0, 2)
    w2 = W_2d3.reshape(C_OUT // 4, 6, C_IN).transpose(1, 0, 2)
    w2 = jnp.concatenate([w2, jnp.zeros((3, C_OUT // 4, C_IN), jnp.float32)], 0)
    w3 = W_d3.reshape(C_OUT // 4, 3, C_IN).transpose(1, 0, 2)
    w3 = jnp.concatenate([w3, jnp.zeros((6, C_OUT // 4, C_IN), jnp.float32)], 0)
    wcat = jnp.concatenate([wf, w2, w3], axis=1)
    wcat = wcat.at[0].add(W1)
    bias = jnp.concatenate([b_full, b_2d3, b_d3]) + b1

    edges = jnp.arange(0, N_HIGH + 1, ROWS_PER_W, dtype=jnp.int32)
    bounds = jnp.searchsorted(up_rows, edges).astype(jnp.int32)
    bounds = jnp.concatenate(
        [bounds, jnp.zeros((NW + LANES + 8 - (NW + 1),), jnp.int32)])
    cols_p = jnp.concatenate([up_cols.astype(jnp.int32),
                              jnp.zeros((CH,), jnp.int32)])
    rows_p = jnp.concatenate([up_rows.astype(jnp.int32),
                              jnp.full((CH,), N_HIGH, jnp.int32)])
    vals_p = jnp.concatenate([up_vals, jnp.zeros((CH,), jnp.float32)])

    x2 = x.reshape(B * N_LOW, C_IN)
    pooled = _pool(x2, cols_p, rows_p, vals_p, bounds)
    h = _matmul(pooled, wcat)
    h2 = h.reshape(B * J * N_HIGH, C_OUT)
    idxt = indices.astype(jnp.int32).T.reshape(J * N_HIGH)
    return _spiral(h2, idxt, bias)


# speedup vs baseline: 13.7787x; 1.8438x over previous
import functools

import jax
import jax.numpy as jnp
from jax import lax
from jax.experimental import pallas as pl
from jax.experimental.pallas import tpu as pltpu
from jax.experimental.pallas import tpu_sc as plsc

B = 4
N_LOW = 8192
N_HIGH = 16384
C_IN = 128
C_OUT = 64
J = 9
NNZ = 3 * N_HIGH

NC, NS, LANES = 2, 16, 16
NW = NC * NS
ROWS_PER_W = N_HIGH // NW
CH = 128
GCH = 64
N_GCH = ROWS_PER_W // GCH

_mesh = lambda: plsc.VectorSubcoreMesh(core_axis_name="c", subcore_axis_name="s")


def _pool_body(x2_hbm, cols_hbm, rows_hbm, vals_hbm, bounds_hbm, out_hbm,
               boundsv, colv, rowv, valv, xg, acc, sem):
    wid = lax.axis_index("s") * NC + lax.axis_index("c")
    base = wid * ROWS_PER_W
    pltpu.sync_copy(bounds_hbm, boundsv)
    s = boundsv[pl.ds(wid, LANES)][0]
    e = boundsv[pl.ds(wid + 1, LANES)][0]
    a = (s // 8) * 8
    nch = (e - a + CH - 1) // CH

    for b in range(B):
        def zero_body(r, _):
            for cb in range(C_IN // LANES):
                acc[r, pl.ds(cb * LANES, LANES)] = jnp.zeros((LANES,), jnp.float32)
            return 0
        lax.fori_loop(0, ROWS_PER_W, zero_body, 0)

        col_off = b * N_LOW

        def chunk_body(ci, _):
            off = a + ci * CH
            pltpu.sync_copy(cols_hbm.at[pl.ds(off, CH)], colv)
            pltpu.sync_copy(rows_hbm.at[pl.ds(off, CH)], rowv.at[pl.ds(0, CH)])
            pltpu.sync_copy(vals_hbm.at[pl.ds(off, CH)], valv.at[pl.ds(0, CH)])
            for q in range(CH // LANES):
                sl = pl.ds(q * LANES, LANES)
                colv[sl] = colv[sl] + col_off
            pltpu.async_copy(x2_hbm.at[colv], xg, sem).wait()

            lo = jnp.maximum(s - off, 0)
            hi = jnp.minimum(e - off, CH)

            def k_body(k, _):
                dst = rowv[pl.ds(k, LANES)][0] - base
                vv = jnp.full((LANES,), valv[pl.ds(k, LANES)][0], jnp.float32)
                for cb in range(C_IN // LANES):
                    sl = pl.ds(cb * LANES, LANES)
                    acc[dst, sl] = acc[dst, sl] + xg[k, sl] * vv
                return 0
            lax.fori_loop(lo, hi, k_body, 0)
            return 0
        lax.fori_loop(0, nch, chunk_body, 0)

        pltpu.sync_copy(acc, out_hbm.at[b, pl.ds(base, ROWS_PER_W)])


def _pool(x2, cols, rows, vals, bounds):
    k = pl.kernel(
        _pool_body,
        out_type=jax.ShapeDtypeStruct((B, N_HIGH, C_IN), jnp.float32),
        mesh=_mesh(),
        scratch_types=[
            pltpu.VMEM((NW + LANES + 8,), jnp.int32),
            pltpu.VMEM((CH,), jnp.int32),
            pltpu.VMEM((CH + LANES,), jnp.int32),
            pltpu.VMEM((CH + LANES,), jnp.float32),
            pltpu.VMEM((CH, C_IN), jnp.float32),
            pltpu.VMEM((ROWS_PER_W, C_IN), jnp.float32),
            pltpu.SemaphoreType.DMA,
        ],
    )
    return k(x2, cols, rows, vals, bounds)


NP = 5
MM_BT = 2048


def _mm_body(p_ref, w_ref, *o_refs):
    res = lax.dot_general(
        p_ref[...], w_ref[...], (((1,), (0,)), ((), ())),
        preferred_element_type=jnp.float32)
    for p5 in range(NP):
        o_refs[p5][...] = res[:, p5 * C_IN:(p5 + 1) * C_IN]


def _matmul(pooled2, w640):
    return pl.pallas_call(
        _mm_body,
        grid=(B * N_HIGH // MM_BT,),
        in_specs=[
            pl.BlockSpec((MM_BT, C_IN), lambda t: (t, 0)),
            pl.BlockSpec((C_IN, NP * C_IN), lambda t: (0, 0)),
        ],
        out_specs=[pl.BlockSpec((MM_BT, C_IN), lambda t: (t, 0))] * NP,
        out_shape=[jax.ShapeDtypeStruct((B * N_HIGH, C_IN), jnp.float32)] * NP,
    )(pooled2, w640)


def _spiral_body(h0, h1, h2, h3, h4, idxt_hbm, bias_hbm, out_hbm,
                 biasv, idxfull, idxv, gbuf, ybuf, sem):
    tables = (h0, h1, h2, h3, h4)
    wid = lax.axis_index("s") * NC + lax.axis_index("c")
    nb = wid * ROWS_PER_W
    pltpu.sync_copy(bias_hbm, biasv)
    for j in range(J):
        pltpu.sync_copy(idxt_hbm.at[pl.ds(j * N_HIGH + nb, ROWS_PER_W)],
                        idxfull.at[pl.ds(j * ROWS_PER_W, ROWS_PER_W)])

    def bc_body(t, _):
        b = t // N_GCH
        c = t % N_GCH
        off = b * N_HIGH
        for j in range(J):
            for q in range(GCH // LANES):
                dsl = pl.ds(j * GCH + q * LANES, LANES)
                ssl = pl.ds(j * ROWS_PER_W + c * GCH + q * LANES, LANES)
                idxv[dsl] = idxfull[ssl] + off
        copies = [
            pltpu.async_copy(
                tables[j // 2].at[idxv.at[pl.ds(j * GCH, GCH)]],
                gbuf.at[j], sem)
            for j in range(J)
        ]
        for cp in copies:
            cp.wait()

        def r_body(r, _):
            for cb in range(C_OUT // LANES):
                sl = pl.ds(cb * LANES, LANES)
                accv = biasv[sl]
                for j in range(J):
                    hsl = pl.ds((j % 2) * C_OUT + cb * LANES, LANES)
                    accv = accv + gbuf[j, r, hsl]
                ybuf[r, sl] = jnp.maximum(accv, 0.0)
            return 0
        lax.fori_loop(0, GCH, r_body, 0)

        pltpu.sync_copy(ybuf, out_hbm.at[b, pl.ds(nb + c * GCH, GCH)])
        return 0

    lax.fori_loop(0, B * N_GCH, bc_body, 0)


def _spiral(hs, idxt, bias):
    k = pl.kernel(
        _spiral_body,
        out_type=jax.ShapeDtypeStruct((B, N_HIGH, C_OUT), jnp.float32),
        mesh=_mesh(),
        scratch_types=[
            pltpu.VMEM((C_OUT,), jnp.float32),
            pltpu.VMEM((J * ROWS_PER_W,), jnp.int32),
            pltpu.VMEM((J * GCH,), jnp.int32),
            pltpu.VMEM((J, GCH, C_IN), jnp.float32),
            pltpu.VMEM((GCH, C_OUT), jnp.float32),
            pltpu.SemaphoreType.DMA,
        ],
        compiler_params=pltpu.CompilerParams(use_tc_tiling_on_sc=False),
    )
    return k(*hs, idxt, bias)


def kernel(x, up_rows, up_cols, up_vals, indices,
           W1, b1, W_d3, b_d3, W_2d3, b_2d3, W_full, b_full):
    wf = W_full.reshape(C_OUT // 2, J, C_IN).transpose(1, 0, 2)
    w2 = W_2d3.reshape(C_OUT // 4, 6, C_IN).transpose(1, 0, 2)
    w2 = jnp.concatenate([w2, jnp.zeros((3, C_OUT // 4, C_IN), jnp.float32)], 0)
    w3 = W_d3.reshape(C_OUT // 4, 3, C_IN).transpose(1, 0, 2)
    w3 = jnp.concatenate([w3, jnp.zeros((6, C_OUT // 4, C_IN), jnp.float32)], 0)
    wcat = jnp.concatenate([wf, w2, w3], axis=1)
    wcat = wcat.at[0].add(W1)
    bias = jnp.concatenate([b_full, b_2d3, b_d3]) + b1

    edges = jnp.arange(0, N_HIGH + 1, ROWS_PER_W, dtype=jnp.int32)
    bounds = jnp.searchsorted(up_rows, edges).astype(jnp.int32)
    bounds = jnp.concatenate(
        [bounds, jnp.zeros((NW + LANES + 8 - (NW + 1),), jnp.int32)])
    cols_p = jnp.concatenate([up_cols.astype(jnp.int32),
                              jnp.zeros((CH,), jnp.int32)])
    rows_p = jnp.concatenate([up_rows.astype(jnp.int32),
                              jnp.full((CH,), N_HIGH, jnp.int32)])
    vals_p = jnp.concatenate([up_vals, jnp.zeros((CH,), jnp.float32)])

    w640 = jnp.pad(wcat.transpose(2, 0, 1).reshape(C_IN, J * C_OUT),
                   ((0, 0), (0, NP * C_IN - J * C_OUT)))

    x2 = x.reshape(B * N_LOW, C_IN)
    pooled = _pool(x2, cols_p, rows_p, vals_p, bounds)
    pooled2 = pooled.reshape(B * N_HIGH, C_IN)
    hs = _matmul(pooled2, w640)
    idxt = indices.astype(jnp.int32).T.reshape(J * N_HIGH)
    return _spiral(hs, idxt, bias)


# speedup vs baseline: 17.3486x; 1.2591x over previous
import functools

import jax
import jax.numpy as jnp
from jax import lax
from jax.experimental import pallas as pl
from jax.experimental.pallas import tpu as pltpu
from jax.experimental.pallas import tpu_sc as plsc

B = 4
N_LOW = 8192
N_HIGH = 16384
C_IN = 128
C_OUT = 64
J = 9
NNZ = 3 * N_HIGH

NC, NS, LANES = 2, 16, 16
NW = NC * NS
ROWS_PER_W = N_HIGH // NW
CH = 128
GCH = 64
N_GCH = ROWS_PER_W // GCH

_mesh = lambda: plsc.VectorSubcoreMesh(core_axis_name="c", subcore_axis_name="s")


NSEG = 4
QHALF = N_HIGH // NSEG
SLAB = QHALF // NS


def _pool_body(x2_hbm, cols_hbm, rows_hbm, vals_hbm, bounds_hbm, out_hbm,
               boundsv, colv, rowv, valv, idxb, xg, zbuf, shared, sem):
    cid = lax.axis_index("c")
    sid = lax.axis_index("s")
    pltpu.sync_copy(bounds_hbm, boundsv)

    def zb_body(r, _):
        for cb in range(C_IN // LANES):
            zbuf[r, pl.ds(cb * LANES, LANES)] = jnp.zeros((LANES,), jnp.float32)
        return 0
    lax.fori_loop(0, SLAB, zb_body, 0)

    def bp_body(t, _):
        b = t // 2
        seg = cid * 2 + (t % 2)
        rbase = seg * QHALF
        s = boundsv[pl.ds(seg * 8, LANES)][0]
        e = boundsv[pl.ds(seg * 8 + 8, LANES)][0]
        a = (s // 8) * 8
        nch = (e - a + CH - 1) // CH
        my_nch = (nch - sid + NS - 1) // NS

        pltpu.sync_copy(zbuf, shared.at[pl.ds(sid * SLAB, SLAB)])
        plsc.subcore_barrier()

        col_off = b * N_LOW

        def chunk_body(i, _):
            off = a + (sid + i * NS) * CH
            pltpu.sync_copy(cols_hbm.at[pl.ds(off, CH)], colv)
            pltpu.sync_copy(rows_hbm.at[pl.ds(off, CH)], rowv)
            pltpu.sync_copy(vals_hbm.at[pl.ds(off, CH)],
                            valv.at[pl.ds(0, CH)])
            for q in range(CH // LANES):
                sl = pl.ds(q * LANES, LANES)
                colv[sl] = colv[sl] + col_off
                idxb[sl] = jnp.clip(rowv[sl] - rbase, 0, QHALF - 1)
            pltpu.async_copy(x2_hbm.at[colv], xg, sem).wait()

            def r_body(r, _):
                kk = off + r
                v = valv[pl.ds(r, LANES)][0]
                v = jnp.where((kk >= s) & (kk < e), v, 0.0)
                vv = jnp.full((LANES,), v, jnp.float32)
                for cb in range(C_IN // LANES):
                    sl = pl.ds(cb * LANES, LANES)
                    xg[r, sl] = xg[r, sl] * vv
                return 0
            lax.fori_loop(0, CH, r_body, 0)

            pltpu.sync_copy(xg, shared.at[idxb], add=True)
            return 0
        lax.fori_loop(0, my_nch, chunk_body, 0)

        plsc.subcore_barrier()
        pltpu.sync_copy(shared.at[pl.ds(sid * SLAB, SLAB)],
                        out_hbm.at[b, pl.ds(rbase + sid * SLAB, SLAB)])
        plsc.subcore_barrier()
        return 0
    lax.fori_loop(0, B * 2, bp_body, 0)


def _pool(x2, cols, rows, vals, bounds):
    k = pl.kernel(
        _pool_body,
        out_type=jax.ShapeDtypeStruct((B, N_HIGH, C_IN), jnp.float32),
        mesh=_mesh(),
        scratch_types=[
            pltpu.VMEM((NW + LANES + 8,), jnp.int32),
            pltpu.VMEM((CH,), jnp.int32),
            pltpu.VMEM((CH,), jnp.int32),
            pltpu.VMEM((CH + LANES,), jnp.float32),
            pltpu.VMEM((CH,), jnp.int32),
            pltpu.VMEM((CH, C_IN), jnp.float32),
            pltpu.VMEM((SLAB, C_IN), jnp.float32),
            pltpu.VMEM_SHARED((QHALF, C_IN), jnp.float32),
            pltpu.SemaphoreType.DMA,
        ],
    )
    return k(x2, cols, rows, vals, bounds)


NP = 5
MM_BT = 2048


def _mm_body(p_ref, w_ref, *o_refs):
    res = lax.dot_general(
        p_ref[...], w_ref[...], (((1,), (0,)), ((), ())),
        preferred_element_type=jnp.float32)
    for p5 in range(NP):
        o_refs[p5][...] = res[:, p5 * C_IN:(p5 + 1) * C_IN]


def _matmul(pooled2, w640):
    return pl.pallas_call(
        _mm_body,
        grid=(B * N_HIGH // MM_BT,),
        in_specs=[
            pl.BlockSpec((MM_BT, C_IN), lambda t: (t, 0)),
            pl.BlockSpec((C_IN, NP * C_IN), lambda t: (0, 0)),
        ],
        out_specs=[pl.BlockSpec((MM_BT, C_IN), lambda t: (t, 0))] * NP,
        out_shape=[jax.ShapeDtypeStruct((B * N_HIGH, C_IN), jnp.float32)] * NP,
    )(pooled2, w640)


def _spiral_body(h0, h1, h2, h3, h4, idxt_hbm, bias_hbm, out_hbm,
                 biasv, idxfull, idxv, gbuf, ybuf, sem):
    tables = (h0, h1, h2, h3, h4)
    wid = lax.axis_index("s") * NC + lax.axis_index("c")
    nb = wid * ROWS_PER_W
    pltpu.sync_copy(bias_hbm, biasv)
    for j in range(J):
        pltpu.sync_copy(idxt_hbm.at[pl.ds(j * N_HIGH + nb, ROWS_PER_W)],
                        idxfull.at[pl.ds(j * ROWS_PER_W, ROWS_PER_W)])

    def bc_body(t, _):
        b = t // N_GCH
        c = t % N_GCH
        off = b * N_HIGH
        for j in range(J):
            for q in range(GCH // LANES):
                dsl = pl.ds(j * GCH + q * LANES, LANES)
                ssl = pl.ds(j * ROWS_PER_W + c * GCH + q * LANES, LANES)
                idxv[dsl] = idxfull[ssl] + off
        copies = [
            pltpu.async_copy(
                tables[j // 2].at[idxv.at[pl.ds(j * GCH, GCH)]],
                gbuf.at[j], sem)
            for j in range(J)
        ]
        for cp in copies:
            cp.wait()

        def r_body(r, _):
            for cb in range(C_OUT // LANES):
                sl = pl.ds(cb * LANES, LANES)
                accv = biasv[sl]
                for j in range(J):
                    hsl = pl.ds((j % 2) * C_OUT + cb * LANES, LANES)
                    accv = accv + gbuf[j, r, hsl]
                ybuf[r, sl] = jnp.maximum(accv, 0.0)
            return 0
        lax.fori_loop(0, GCH, r_body, 0)

        pltpu.sync_copy(ybuf, out_hbm.at[b, pl.ds(nb + c * GCH, GCH)])
        return 0

    lax.fori_loop(0, B * N_GCH, bc_body, 0)


def _spiral(hs, idxt, bias):
    k = pl.kernel(
        _spiral_body,
        out_type=jax.ShapeDtypeStruct((B, N_HIGH, C_OUT), jnp.float32),
        mesh=_mesh(),
        scratch_types=[
            pltpu.VMEM((C_OUT,), jnp.float32),
            pltpu.VMEM((J * ROWS_PER_W,), jnp.int32),
            pltpu.VMEM((J * GCH,), jnp.int32),
            pltpu.VMEM((J, GCH, C_IN), jnp.float32),
            pltpu.VMEM((GCH, C_OUT), jnp.float32),
            pltpu.SemaphoreType.DMA,
        ],
        compiler_params=pltpu.CompilerParams(use_tc_tiling_on_sc=False),
    )
    return k(*hs, idxt, bias)


def kernel(x, up_rows, up_cols, up_vals, indices,
           W1, b1, W_d3, b_d3, W_2d3, b_2d3, W_full, b_full):
    wf = W_full.reshape(C_OUT // 2, J, C_IN).transpose(1, 0, 2)
    w2 = W_2d3.reshape(C_OUT // 4, 6, C_IN).transpose(1, 0, 2)
    w2 = jnp.concatenate([w2, jnp.zeros((3, C_OUT // 4, C_IN), jnp.float32)], 0)
    w3 = W_d3.reshape(C_OUT // 4, 3, C_IN).transpose(1, 0, 2)
    w3 = jnp.concatenate([w3, jnp.zeros((6, C_OUT // 4, C_IN), jnp.float32)], 0)
    wcat = jnp.concatenate([wf, w2, w3], axis=1)
    wcat = wcat.at[0].add(W1)
    bias = jnp.concatenate([b_full, b_2d3, b_d3]) + b1

    edges = jnp.arange(0, N_HIGH + 1, ROWS_PER_W, dtype=jnp.int32)
    bounds = jnp.searchsorted(up_rows, edges).astype(jnp.int32)
    bounds = jnp.concatenate(
        [bounds, jnp.zeros((NW + LANES + 8 - (NW + 1),), jnp.int32)])
    cols_p = jnp.concatenate([up_cols.astype(jnp.int32),
                              jnp.zeros((CH,), jnp.int32)])
    rows_p = jnp.concatenate([up_rows.astype(jnp.int32),
                              jnp.full((CH,), N_HIGH, jnp.int32)])
    vals_p = jnp.concatenate([up_vals, jnp.zeros((CH,), jnp.float32)])

    w640 = jnp.pad(wcat.transpose(2, 0, 1).reshape(C_IN, J * C_OUT),
                   ((0, 0), (0, NP * C_IN - J * C_OUT)))

    x2 = x.reshape(B * N_LOW, C_IN)
    pooled = _pool(x2, cols_p, rows_p, vals_p, bounds)
    pooled2 = pooled.reshape(B * N_HIGH, C_IN)
    hs = _matmul(pooled2, w640)
    idxt = indices.astype(jnp.int32).T.reshape(J * N_HIGH)
    return _spiral(hs, idxt, bias)
